# Initial kernel scaffold; baseline (speedup 1.0000x reference)
import jax
import jax.numpy as jnp
from jax.experimental import pallas as pl


def kernel(x, batch):
    raise NotImplementedError("write your pallas kernel here")


# speedup vs baseline: 5.6373x; 5.6373x over previous
import functools

import jax
import jax.numpy as jnp
from jax import lax
from jax.experimental import pallas as pl
from jax.experimental.pallas import tpu as pltpu
from jax.experimental.pallas import tpu_sc as plsc

N = 100000
D = 128
S = 256
NC = 2
NS = 16
NW = NC * NS
CH = 128
NFULL = N // CH
TAIL = N - NFULL * CH
TAIL_OFF = NFULL * CH
EXTRA = NFULL % NW


def _sc_partials(x, batch, z2, z1):
    mesh = plsc.VectorSubcoreMesh(core_axis_name="c", subcore_axis_name="s")

    @functools.partial(
        pl.kernel,
        out_type=(
            jax.ShapeDtypeStruct((NC * S, D), jnp.float32),
            jax.ShapeDtypeStruct((NC * S,), jnp.float32),
        ),
        mesh=mesh,
        scratch_types=[
            pltpu.VMEM((CH, D), jnp.float32),
            pltpu.VMEM((CH,), jnp.int32),
            pltpu.VMEM((TAIL,), jnp.int32),
            pltpu.VMEM((CH,), jnp.float32),
            pltpu.VMEM((S,), jnp.float32),
            pltpu.VMEM_SHARED((S, D), jnp.float32),
            pltpu.VMEM_SHARED((S,), jnp.float32),
        ],
    )
    def k(x_hbm, b_hbm, z2_hbm, z1_hbm, psum_hbm, pcnt_hbm,
          xbuf, idxbuf, tidxbuf, ones, cntbuf, acc_sh, cnt_sh):
        cid = lax.axis_index("c")
        sid = lax.axis_index("s")
        wid = cid * NS + sid

        pltpu.sync_copy(z2_hbm.at[pl.ds(sid * (S // NS), S // NS), :],
                        xbuf.at[pl.ds(0, S // NS), :])
        pltpu.sync_copy(xbuf.at[pl.ds(0, S // NS), :],
                        acc_sh.at[pl.ds(sid * (S // NS), S // NS), :])

        @pl.when(sid == 0)
        def _():
            pltpu.sync_copy(z1_hbm, cntbuf)
            pltpu.sync_copy(cntbuf, cnt_sh)

        for i in range(CH // 16):
            ones[pl.ds(i * 16, 16)] = jnp.ones((16,), jnp.float32)

        plsc.subcore_barrier()

        nchunks = jnp.where(wid < EXTRA, NFULL // NW + 1, NFULL // NW)

        def body(j, carry):
            off = (wid + j * NW) * CH
            off = pl.multiple_of(off, CH)
            pltpu.sync_copy(b_hbm.at[pl.ds(off, CH)], idxbuf)
            pltpu.sync_copy(x_hbm.at[pl.ds(off, CH), :], xbuf)
            pltpu.sync_copy(xbuf, acc_sh.at[idxbuf], add=True)
            pltpu.sync_copy(ones, cnt_sh.at[idxbuf], add=True)
            return carry

        lax.fori_loop(0, nchunks, body, 0)

        @pl.when(wid == NW - 1)
        def _():
            pltpu.sync_copy(b_hbm.at[pl.ds(TAIL_OFF, TAIL)], tidxbuf)
            pltpu.sync_copy(x_hbm.at[pl.ds(TAIL_OFF, TAIL), :],
                            xbuf.at[pl.ds(0, TAIL), :])
            pltpu.sync_copy(xbuf.at[pl.ds(0, TAIL), :],
                            acc_sh.at[tidxbuf], add=True)
            pltpu.sync_copy(ones.at[pl.ds(0, TAIL)],
                            cnt_sh.at[tidxbuf], add=True)

        plsc.subcore_barrier()

        pltpu.sync_copy(acc_sh.at[pl.ds(sid * (S // NS), S // NS), :],
                        xbuf.at[pl.ds(0, S // NS), :])
        pltpu.sync_copy(xbuf.at[pl.ds(0, S // NS), :],
                        psum_hbm.at[pl.ds(cid * S + sid * (S // NS), S // NS), :])

        @pl.when(sid == 0)
        def _():
            pltpu.sync_copy(cnt_sh, cntbuf)
            pltpu.sync_copy(cntbuf, pcnt_hbm.at[pl.ds(cid * S, S)])

    return k(x, batch, z2, z1)


def _merge_kernel(ps_ref, pc_ref, o_ref):
    sums = ps_ref[0:S, :] + ps_ref[S:2 * S, :]
    cnts = pc_ref[0, 0:S] + pc_ref[0, S:2 * S]
    o_ref[...] = sums / jnp.maximum(cnts, 1.0)[:, None]


@jax.jit
def kernel(x, batch):
    batch = batch.astype(jnp.int32)
    z2 = jnp.zeros((S, D), jnp.float32)
    z1 = jnp.zeros((S,), jnp.float32)
    psum, pcnt = _sc_partials(x, batch, z2, z1)
    return pl.pallas_call(
        _merge_kernel,
        out_shape=jax.ShapeDtypeStruct((S, D), jnp.float32),
    )(psum, pcnt.reshape(1, NC * S))


# speedup vs baseline: 8.0885x; 1.4348x over previous
import functools

import jax
import jax.numpy as jnp
from jax import lax
from jax.experimental import pallas as pl
from jax.experimental.pallas import tpu as pltpu
from jax.experimental.pallas import tpu_sc as plsc

N = 100000
D = 128
S = 256
NC = 2
NS = 16
NW = NC * NS
CH = 128
NFULL = N // CH
TAIL = N - NFULL * CH
TAIL_OFF = NFULL * CH
EXTRA = NFULL % NW
NMAX = NFULL // NW + 1
NBUF = 4


def _sc_partials(x, batch, z2, z1):
    mesh = plsc.VectorSubcoreMesh(core_axis_name="c", subcore_axis_name="s")

    @functools.partial(
        pl.kernel,
        out_type=(
            jax.ShapeDtypeStruct((NC * S, D), jnp.float32),
            jax.ShapeDtypeStruct((NC * S,), jnp.float32),
        ),
        mesh=mesh,
        scratch_types=[
            pltpu.VMEM((NBUF, CH, D), jnp.float32),
            pltpu.VMEM((NBUF, CH), jnp.int32),
            pltpu.VMEM((TAIL,), jnp.int32),
            pltpu.VMEM((S // NS, D), jnp.float32),
            pltpu.VMEM((CH,), jnp.float32),
            pltpu.VMEM((S,), jnp.float32),
            pltpu.VMEM_SHARED((S, D), jnp.float32),
            pltpu.VMEM_SHARED((S,), jnp.float32),
            pltpu.SemaphoreType.DMA((NBUF,)),
            pltpu.SemaphoreType.DMA((NBUF,)),
            pltpu.SemaphoreType.DMA((NBUF,)),
            pltpu.SemaphoreType.DMA((NBUF,)),
        ],
    )
    def k(x_hbm, b_hbm, z2_hbm, z1_hbm, psum_hbm, pcnt_hbm,
          xbuf, idxbuf, tidxbuf, stg, ones, cntbuf, acc_sh, cnt_sh,
          sx, si, ss, sc):
        cid = lax.axis_index("c")
        sid = lax.axis_index("s")
        wid = cid * NS + sid

        pltpu.sync_copy(z2_hbm.at[pl.ds(sid * (S // NS), S // NS), :], stg)
        pltpu.sync_copy(stg, acc_sh.at[pl.ds(sid * (S // NS), S // NS), :])

        @pl.when(sid == 0)
        def _():
            pltpu.sync_copy(z1_hbm, cntbuf)
            pltpu.sync_copy(cntbuf, cnt_sh)

        for i in range(CH // 16):
            ones[pl.ds(i * 16, 16)] = jnp.ones((16,), jnp.float32)

        plsc.subcore_barrier()

        nchunks = jnp.where(wid < EXTRA, NFULL // NW + 1, NFULL // NW)

        def start_in(j, b):
            off = (wid + j * NW) * CH
            off = pl.multiple_of(off, CH)
            pltpu.async_copy(b_hbm.at[pl.ds(off, CH)], idxbuf.at[b], si.at[b])
            pltpu.async_copy(x_hbm.at[pl.ds(off, CH), :], xbuf.at[b], sx.at[b])

        for b in range(NBUF):
            start_in(b, b)

        for j in range(NMAX):
            b = j % NBUF

            @pl.when(j < nchunks)
            def _(j=j, b=b):
                pltpu.make_async_copy(b_hbm.at[pl.ds(0, CH)], idxbuf.at[b],
                                      si.at[b]).wait()
                pltpu.make_async_copy(x_hbm.at[pl.ds(0, CH), :], xbuf.at[b],
                                      sx.at[b]).wait()
                pltpu.async_copy(xbuf.at[b], acc_sh.at[idxbuf.at[b]], ss.at[b],
                                 add=True)
                pltpu.async_copy(ones, cnt_sh.at[idxbuf.at[b]], sc.at[b],
                                 add=True)
                jp = j + NBUF

                @pl.when(jp < nchunks)
                def _():
                    pltpu.make_async_copy(xbuf.at[b],
                                          acc_sh.at[idxbuf.at[b]],
                                          ss.at[b]).wait()
                    pltpu.make_async_copy(ones, cnt_sh.at[idxbuf.at[b]],
                                          sc.at[b]).wait()
                    start_in(jp, b)

        for j in range(NMAX):
            b = j % NBUF

            @pl.when((j + NBUF >= nchunks) & (j < nchunks))
            def _(b=b):
                pltpu.make_async_copy(xbuf.at[b], acc_sh.at[idxbuf.at[b]],
                                      ss.at[b]).wait()
                pltpu.make_async_copy(ones, cnt_sh.at[idxbuf.at[b]],
                                      sc.at[b]).wait()

        @pl.when(wid == NW - 1)
        def _():
            pltpu.sync_copy(b_hbm.at[pl.ds(TAIL_OFF, TAIL)], tidxbuf)
            pltpu.sync_copy(x_hbm.at[pl.ds(TAIL_OFF, TAIL), :],
                            xbuf.at[0, pl.ds(0, TAIL), :])
            pltpu.sync_copy(xbuf.at[0, pl.ds(0, TAIL), :],
                            acc_sh.at[tidxbuf], add=True)
            pltpu.sync_copy(ones.at[pl.ds(0, TAIL)],
                            cnt_sh.at[tidxbuf], add=True)

        plsc.subcore_barrier()

        pltpu.sync_copy(acc_sh.at[pl.ds(sid * (S // NS), S // NS), :], stg)
        pltpu.sync_copy(stg,
                        psum_hbm.at[pl.ds(cid * S + sid * (S // NS), S // NS), :])

        @pl.when(sid == 0)
        def _():
            pltpu.sync_copy(cnt_sh, cntbuf)
            pltpu.sync_copy(cntbuf, pcnt_hbm.at[pl.ds(cid * S, S)])

    return k(x, batch, z2, z1)


def _merge_kernel(ps_ref, pc_ref, o_ref):
    sums = ps_ref[0:S, :] + ps_ref[S:2 * S, :]
    cnts = pc_ref[0, 0:S] + pc_ref[0, S:2 * S]
    o_ref[...] = sums / jnp.maximum(cnts, 1.0)[:, None]


@jax.jit
def kernel(x, batch):
    batch = batch.astype(jnp.int32)
    z2 = jnp.zeros((S, D), jnp.float32)
    z1 = jnp.zeros((S,), jnp.float32)
    psum, pcnt = _sc_partials(x, batch, z2, z1)
    return pl.pallas_call(
        _merge_kernel,
        out_shape=jax.ShapeDtypeStruct((S, D), jnp.float32),
    )(psum, pcnt.reshape(1, NC * S))
